# Initial kernel scaffold; baseline (speedup 1.0000x reference)
import functools

import jax
import jax.numpy as jnp
from jax import lax
from jax.experimental import pallas as pl
from jax.experimental.pallas import tpu as pltpu
from jax.experimental.pallas import tpu_sc as plsc

_B, _T, _D = 4096, 200, 64
_N = _B * _T
_NC, _NS = 2, 16
_NW = _NC * _NS
_RPW = _N // _NW
_C = 128
_NCHUNK = _RPW // _C
_LANES = 16


def _emb_body(x_hbm, pi_hbm, val_tab, pos_tab, out_hbm,
              xi_v, pi_v, val_v, pos_v, sem_val, sem_pos):
    wid = lax.axis_index("s") * _NC + lax.axis_index("c")
    base0 = wid * _RPW

    def chunk(i, carry):
        base = base0 + i * _C
        pltpu.sync_copy(x_hbm.at[pl.ds(base, _C)], xi_v)
        pltpu.sync_copy(pi_hbm.at[pl.ds(base, _C)], pi_v)
        cv = pltpu.async_copy(val_tab.at[xi_v], val_v, sem_val)
        cp = pltpu.async_copy(pos_tab.at[pi_v], pos_v, sem_pos)
        cv.wait()
        cp.wait()

        def addrow(r, c2):
            for c4 in range(_D // _LANES):
                sl = pl.ds(c4 * _LANES, _LANES)
                val_v[r, sl] = val_v[r, sl] + pos_v[r, sl]
            return c2

        lax.fori_loop(0, _C, addrow, 0)
        pltpu.sync_copy(val_v, out_hbm.at[pl.ds(base, _C)])
        return carry

    lax.fori_loop(0, _NCHUNK, chunk, 0)


@jax.jit
def _emb(xf, pf, value_table, pos_table):
    f = pl.kernel(
        _emb_body,
        out_type=jax.ShapeDtypeStruct((_N, _D), jnp.float32),
        mesh=plsc.VectorSubcoreMesh(
            core_axis_name="c", subcore_axis_name="s",
            num_cores=_NC, num_subcores=_NS),
        scratch_types=[
            pltpu.VMEM((_C,), jnp.int32),
            pltpu.VMEM((_C,), jnp.int32),
            pltpu.VMEM((_C, _D), jnp.float32),
            pltpu.VMEM((_C, _D), jnp.float32),
            pltpu.SemaphoreType.DMA,
            pltpu.SemaphoreType.DMA,
        ],
    )
    return f(xf, pf, value_table, pos_table)


def kernel(x, pos_idx, value_table, pos_table):
    xf = x.reshape(_N)
    pf = pos_idx.reshape(_N)
    out = _emb(xf, pf, value_table, pos_table)
    return out.reshape(_B, _T, _D)


# speedup vs baseline: 1.9529x; 1.9529x over previous
import functools

import jax
import jax.numpy as jnp
from jax import lax
from jax.experimental import pallas as pl
from jax.experimental.pallas import tpu as pltpu
from jax.experimental.pallas import tpu_sc as plsc

_B, _T, _D = 4096, 200, 64
_N = _B * _T
_NC, _NS = 2, 16
_NW = _NC * _NS
_RPW = _N // _NW
_C = 128
_NCHUNK = _RPW // _C
_LANES = 16


def _emb_body(x_hbm, pi_hbm, val_tab, pos_tab, out_hbm,
              xi_v, pi_v, val_v, pos_v, sem_val, sem_pos):
    wid = lax.axis_index("s") * _NC + lax.axis_index("c")
    base0 = wid * _RPW

    def chunk(i, carry):
        base = base0 + i * _C
        pltpu.sync_copy(x_hbm.at[pl.ds(base, _C)], xi_v)
        pltpu.sync_copy(pi_hbm.at[pl.ds(base, _C)], pi_v)
        cv = pltpu.async_copy(val_tab.at[xi_v], val_v, sem_val)
        cp = pltpu.async_copy(pos_tab.at[pi_v], pos_v, sem_pos)
        cv.wait()
        cp.wait()

        def addrow(r, c2):
            for c4 in range(_D // _LANES):
                sl = pl.ds(c4 * _LANES, _LANES)
                val_v[r, sl] = val_v[r, sl] + pos_v[r, sl]
            return c2

        lax.fori_loop(0, _C, addrow, 0)
        pltpu.sync_copy(val_v, out_hbm.at[pl.ds(base, _C)])
        return carry

    lax.fori_loop(0, _NCHUNK, chunk, 0)


@jax.jit
def _emb(xf, pf, value_table, pos_table):
    f = pl.kernel(
        _emb_body,
        out_type=jax.ShapeDtypeStruct((_N, _D), jnp.float32),
        mesh=plsc.VectorSubcoreMesh(
            core_axis_name="c", subcore_axis_name="s",
            num_cores=_NC, num_subcores=_NS),
        scratch_types=[
            pltpu.VMEM((_C,), jnp.int32),
            pltpu.VMEM((_C,), jnp.int32),
            pltpu.VMEM((_C, _D), jnp.float32),
            pltpu.VMEM((_C, _D), jnp.float32),
            pltpu.SemaphoreType.DMA,
            pltpu.SemaphoreType.DMA,
        ],
        compiler_params=pltpu.CompilerParams(use_tc_tiling_on_sc=False),
    )
    return f(xf, pf, value_table, pos_table)


def kernel(x, pos_idx, value_table, pos_table):
    xf = x.reshape(_N)
    pf = pos_idx.reshape(_N)
    out = _emb(xf, pf, value_table, pos_table)
    return out.reshape(_B, _T, _D)


# speedup vs baseline: 2.8355x; 1.4519x over previous
import jax
import jax.numpy as jnp
from jax import lax
from jax.experimental import pallas as pl
from jax.experimental.pallas import tpu as pltpu
from jax.experimental.pallas import tpu_sc as plsc

_B, _T, _D = 4096, 200, 64
_N = _B * _T
_CTX = 200
_NC, _NS = 2, 16
_NW = _NC * _NS
_RPW = _N // _NW
_C = 64
_U = 4
_GROUP = 2 * _U * _C
_NBODY = _RPW // _GROUP


def _emb_body(x_hbm, pi_hbm, val_tab, pos_tab, out_hbm,
              xi_all, pi_all, pos_vt, bufs, sem_gv, sem_ga, sem_st):
    wid = lax.axis_index("s") * _NC + lax.axis_index("c")
    base0 = wid * _RPW

    pltpu.sync_copy(x_hbm.at[pl.ds(base0, _RPW)], xi_all)
    pltpu.sync_copy(pi_hbm.at[pl.ds(base0, _RPW)], pi_all)
    pl.when(lax.axis_index("s") == 0)(lambda: pltpu.sync_copy(pos_tab, pos_vt))
    plsc.subcore_barrier()

    def off(k, h, u):
        return k * _GROUP + h * (_U * _C) + u * _C

    def gv(k, h, u):
        pltpu.async_copy(
            val_tab.at[xi_all.at[pl.ds(off(k, h, u), _C)]],
            bufs.at[h, u], sem_gv.at[h, u])

    def gv_wait(h, u):
        pltpu.make_async_copy(
            val_tab.at[xi_all.at[pl.ds(0, _C)]],
            bufs.at[h, u], sem_gv.at[h, u]).wait()

    def ga(k, h, u):
        pltpu.async_copy(
            pos_vt.at[pi_all.at[pl.ds(off(k, h, u), _C)]],
            bufs.at[h, u], sem_ga.at[h, u], add=True)

    def ga_wait(h, u):
        pltpu.make_async_copy(
            pos_vt.at[pi_all.at[pl.ds(0, _C)]],
            bufs.at[h, u], sem_ga.at[h, u]).wait()

    def st(k, h, u):
        pltpu.async_copy(
            bufs.at[h, u],
            out_hbm.at[pl.ds(base0 + off(k, h, u), _C)],
            sem_st.at[h, u])

    def st_wait(h, u):
        pltpu.make_async_copy(
            bufs.at[h, u],
            out_hbm.at[pl.ds(base0, _C)], sem_st.at[h, u]).wait()

    for u in range(_U):
        gv(0, 0, u)

    def body(k, carry):
        for u in range(_U):
            gv_wait(0, u)
            ga(k, 0, u)
        for u in range(_U):
            pl.when(k > 0)(lambda u=u: st_wait(1, u))
            gv(k, 1, u)
        for u in range(_U):
            ga_wait(0, u)
            st(k, 0, u)
        for u in range(_U):
            gv_wait(1, u)
            ga(k, 1, u)
        for u in range(_U):
            st_wait(0, u)
            pl.when(k < _NBODY - 1)(lambda u=u: gv(k + 1, 0, u))
        for u in range(_U):
            ga_wait(1, u)
            st(k, 1, u)
        return carry

    lax.fori_loop(0, _NBODY, body, 0)

    for u in range(_U):
        st_wait(1, u)


@jax.jit
def _emb(xf, pf, value_table, pos_table):
    f = pl.kernel(
        _emb_body,
        out_type=jax.ShapeDtypeStruct((_N, _D), jnp.float32),
        mesh=plsc.VectorSubcoreMesh(
            core_axis_name="c", subcore_axis_name="s",
            num_cores=_NC, num_subcores=_NS),
        scratch_types=[
            pltpu.VMEM((_RPW,), jnp.int32),
            pltpu.VMEM((_RPW,), jnp.int32),
            pltpu.VMEM_SHARED((_CTX, _D), jnp.float32),
            pltpu.VMEM((2, _U, _C, _D), jnp.float32),
            pltpu.SemaphoreType.DMA((2, _U)),
            pltpu.SemaphoreType.DMA((2, _U)),
            pltpu.SemaphoreType.DMA((2, _U)),
        ],
        compiler_params=pltpu.CompilerParams(use_tc_tiling_on_sc=False),
    )
    return f(xf, pf, value_table, pos_table)


def kernel(x, pos_idx, value_table, pos_table):
    xf = x.reshape(_N)
    pf = pos_idx.reshape(_N)
    out = _emb(xf, pf, value_table, pos_table)
    return out.reshape(_B, _T, _D)


# speedup vs baseline: 2.8383x; 1.0010x over previous
import jax
import jax.numpy as jnp
from jax import lax
from jax.experimental import pallas as pl
from jax.experimental.pallas import tpu as pltpu
from jax.experimental.pallas import tpu_sc as plsc

_B, _T, _D = 4096, 200, 64
_N = _B * _T
_CTX = 200
_NC, _NS = 2, 16
_NW = _NC * _NS
_RPW = _N // _NW
_C = 64
_U = 4
_GROUP = 2 * _U * _C
_NBODY = _RPW // _GROUP


def _emb_body(x_hbm, pi_hbm, val_tab, pos_tab, out_hbm,
              xi_all, pi_all, pos_vt, bufs, sem_gv, sem_ga, sem_st):
    wid = lax.axis_index("s") * _NC + lax.axis_index("c")
    base0 = wid * _RPW

    pltpu.sync_copy(x_hbm.at[pl.ds(base0, _RPW)], xi_all)
    pltpu.sync_copy(pi_hbm.at[pl.ds(base0, _RPW)], pi_all)
    pl.when(lax.axis_index("s") == 0)(lambda: pltpu.sync_copy(pos_tab, pos_vt))
    plsc.subcore_barrier()

    def off(k, h, u):
        return k * _GROUP + h * (_U * _C) + u * _C

    def gv(k, h, u):
        pltpu.async_copy(
            val_tab.at[xi_all.at[pl.ds(off(k, h, u), _C)]],
            bufs.at[h, u], sem_gv.at[h, u])

    def gv_wait(h, u):
        pltpu.make_async_copy(
            val_tab.at[xi_all.at[pl.ds(0, _C)]],
            bufs.at[h, u], sem_gv.at[h, u]).wait()

    def ga(k, h, u):
        pltpu.async_copy(
            pos_vt.at[pi_all.at[pl.ds(off(k, h, u), _C)]],
            bufs.at[h, u], sem_ga.at[h, u], add=True)

    def ga_wait(h, u):
        pltpu.make_async_copy(
            pos_vt.at[pi_all.at[pl.ds(0, _C)]],
            bufs.at[h, u], sem_ga.at[h, u]).wait()

    def st(k, h, u):
        pltpu.async_copy(
            bufs.at[h, u],
            out_hbm.at[pl.ds(base0 + off(k, h, u), _C)],
            sem_st.at[h, u])

    def st_wait(h, u):
        pltpu.make_async_copy(
            bufs.at[h, u],
            out_hbm.at[pl.ds(base0, _C)], sem_st.at[h, u]).wait()

    for u in range(_U):
        gv(0, 0, u)

    def body(k, carry):
        for u in range(_U):
            gv_wait(0, u)
            ga(k, 0, u)
        for u in range(_U):
            pl.when(k > 0)(lambda u=u: st_wait(1, u))
            gv(k, 1, u)
        for u in range(_U):
            ga_wait(0, u)
            st(k, 0, u)
        for u in range(_U):
            gv_wait(1, u)
            ga(k, 1, u)
        for u in range(_U):
            st_wait(0, u)
            pl.when(k < _NBODY - 1)(lambda u=u: gv(k + 1, 0, u))
        for u in range(_U):
            ga_wait(1, u)
            st(k, 1, u)
        return carry

    lax.fori_loop(0, _NBODY, body, 0)

    for u in range(_U):
        st_wait(1, u)


@jax.jit
def _emb(xf, pf, value_table, pos_table):
    f = pl.kernel(
        _emb_body,
        out_type=jax.ShapeDtypeStruct((_N, _D), jnp.float32),
        mesh=plsc.VectorSubcoreMesh(
            core_axis_name="c", subcore_axis_name="s",
            num_cores=_NC, num_subcores=_NS),
        scratch_types=[
            pltpu.VMEM((_RPW,), jnp.int32),
            pltpu.VMEM((_RPW,), jnp.int32),
            pltpu.VMEM_SHARED((_CTX, _D), jnp.float32),
            pltpu.VMEM((2, _U, _C, _D), jnp.float32),
            pltpu.SemaphoreType.DMA((2, _U)),
            pltpu.SemaphoreType.DMA((2, _U)),
            pltpu.SemaphoreType.DMA((2, _U)),
        ],
        compiler_params=pltpu.CompilerParams(
            use_tc_tiling_on_sc=False, skip_device_barrier=True),
    )
    return f(xf, pf, value_table, pos_table)


def kernel(x, pos_idx, value_table, pos_table):
    xf = x.reshape(_N)
    pf = pos_idx.reshape(_N)
    out = _emb(xf, pf, value_table, pos_table)
    return out.reshape(_B, _T, _D)
